# Initial kernel scaffold; baseline (speedup 1.0000x reference)
import jax
import jax.numpy as jnp
from jax.experimental import pallas as pl


def kernel(img1, img2, flow, valid_flow_mask):
    raise NotImplementedError("write your pallas kernel here")


# speedup vs baseline: 282.2753x; 282.2753x over previous
import jax
import jax.numpy as jnp
from jax.experimental import pallas as pl

_IMG_R0 = 204
_IMG_NR = 584
_IMG_C0 = 409
_IMG_NC = 1152
_FLW_R0 = 205
_FLW_NR = 576
_FLW_C0 = 410
_FLW_NC = 1056

def _sx_rel(xo):
    xa = 512 + xo
    r = xa % 10
    return 8 * (xa // 10) + r - (1 if r >= 4 else 0) - (1 if r >= 8 else 0) - _FLW_C0

_FLW_WSTART = tuple(_sx_rel(128 * t) for t in range(10))


def _img_body(a_ref, b_ref, oa_ref, ob_ref):
    fys = (0.7, 0.5, 0.3, 0.1, 0.9)
    cs = (0, 1, 2, 3, 3)
    lane = jax.lax.broadcasted_iota(jnp.int32, (720, 128), 1)
    for src, dst in ((a_ref, oa_ref), (b_ref, ob_ref)):
        x = src[0]
        a3 = x[0:576].reshape(144, 4, _IMG_NC)
        b3 = x[1:577].reshape(144, 4, _IMG_NC)
        ps = [a3[:, c, :] * (1.0 - f) + b3[:, c, :] * f for c, f in zip(cs, fys)]
        v = jnp.stack(ps, axis=1).reshape(720, _IMG_NC)
        for t in range(10):
            s = (512 * t + 2) // 5
            win = v[:, s:s + 128]
            xabs = 512 + 128 * t + lane
            q = (8 * xabs - 1) // 10
            jrel = q - (_IMG_C0 + s)
            fx = ((8 * xabs - 1) - 10 * q).astype(jnp.float32) * 0.1
            g0 = jnp.take_along_axis(win, jrel, axis=1)
            g1 = jnp.take_along_axis(win, jrel + 1, axis=1)
            dst[0, :, 128 * t:128 * (t + 1)] = g0 * (1.0 - fx) + g1 * fx


def _flow_body(f_ref, m_ref, of_ref, ov_ref):
    c10 = (0, 0, 1, 2, 3, 4, 5, 5, 6, 7)
    x = f_ref[0] * 1.25
    m = m_ref[...].astype(jnp.float32)
    x3 = x.reshape(72, 8, _FLW_NC)
    m3 = m.reshape(72, 8, _FLW_NC)
    vx = jnp.stack([x3[:, c, :] for c in c10], axis=1).reshape(720, _FLW_NC)
    vm = jnp.stack([m3[:, c, :] for c in c10], axis=1).reshape(720, _FLW_NC)
    lane = jax.lax.broadcasted_iota(jnp.int32, (720, 128), 1)
    yo = jax.lax.broadcasted_iota(jnp.int32, (720, 128), 0)
    ry = yo % 10
    rowmask = jnp.logical_and(ry != 1, ry != 7)
    for t in range(10):
        w = _FLW_WSTART[t]
        xa = 512 + 128 * t + lane
        r = xa % 10
        sx = (8 * (xa // 10) + r - (r >= 4).astype(jnp.int32)
              - (r >= 8).astype(jnp.int32))
        jrel = sx - (_FLW_C0 + w)
        colmask = jnp.logical_and(r != 3, r != 7)
        keep = jnp.logical_and(colmask, rowmask)
        gx = jnp.take_along_axis(vx[:, w:w + 128], jrel, axis=1)
        gm = jnp.take_along_axis(vm[:, w:w + 128], jrel, axis=1)
        of_ref[0, :, 128 * t:128 * (t + 1)] = gx * gm * keep.astype(jnp.float32)
        ov_ref[:, 128 * t:128 * (t + 1)] = jnp.logical_and(gm > 0.5, keep)


def _run_images(i1, i2):
    return pl.pallas_call(
        _img_body,
        grid=(3,),
        in_specs=[pl.BlockSpec((1, _IMG_NR, _IMG_NC), lambda c: (c, 0, 0))] * 2,
        out_specs=[pl.BlockSpec((1, 720, 1280), lambda c: (c, 0, 0))] * 2,
        out_shape=[jax.ShapeDtypeStruct((3, 720, 1280), jnp.float32)] * 2,
    )(i1, i2)


def _run_flow(fl, mk):
    return pl.pallas_call(
        _flow_body,
        grid=(2,),
        in_specs=[
            pl.BlockSpec((1, _FLW_NR, _FLW_NC), lambda c: (c, 0, 0)),
            pl.BlockSpec((_FLW_NR, _FLW_NC), lambda c: (0, 0)),
        ],
        out_specs=[
            pl.BlockSpec((1, 720, 1280), lambda c: (c, 0, 0)),
            pl.BlockSpec((720, 1280), lambda c: (0, 0)),
        ],
        out_shape=[
            jax.ShapeDtypeStruct((2, 720, 1280), jnp.float32),
            jax.ShapeDtypeStruct((720, 1280), jnp.bool_),
        ],
    )(fl, mk)


def kernel(img1, img2, flow, valid_flow_mask):
    i1 = jax.lax.slice(img1, (0, _IMG_R0, _IMG_C0),
                       (3, _IMG_R0 + _IMG_NR, _IMG_C0 + _IMG_NC))
    i2 = jax.lax.slice(img2, (0, _IMG_R0, _IMG_C0),
                       (3, _IMG_R0 + _IMG_NR, _IMG_C0 + _IMG_NC))
    fl = jax.lax.slice(flow, (0, _FLW_R0, _FLW_C0),
                       (2, _FLW_R0 + _FLW_NR, _FLW_C0 + _FLW_NC))
    mk = jax.lax.slice(valid_flow_mask, (_FLW_R0, _FLW_C0),
                       (_FLW_R0 + _FLW_NR, _FLW_C0 + _FLW_NC))
    o1, o2 = _run_images(i1, i2)
    fo, vo = _run_flow(fl, mk)
    return o1, o2, fo, vo


# speedup vs baseline: 369.0157x; 1.3073x over previous
import jax
import jax.numpy as jnp
from jax.experimental import pallas as pl
from jax.experimental.pallas import tpu as pltpu

_IMG_R0 = 204
_IMG_NR = 584
_IMG_C0 = 409
_IMG_NC = 1152
_FLW_R0 = 205
_FLW_NR = 584
_FLW_C0 = 410
_FLW_NC = 1056


def _iy_rel(yo):
    return (8 * (yo + 256) - 1) // 10 - _IMG_R0


def _sy_rel(yo):
    y = yo + 256
    r = y % 10
    return 8 * (y // 10) + r - (1 if r >= 4 else 0) - (1 if r >= 8 else 0) - _FLW_R0


def _sx_rel(xo):
    xa = 512 + xo
    r = xa % 10
    return 8 * (xa // 10) + r - (1 if r >= 4 else 0) - (1 if r >= 8 else 0) - _FLW_C0


_FLW_WSTART = tuple(_sx_rel(128 * t) for t in range(10))


def _img_body(a_ref, b_ref, oa_ref, ob_ref, v_ref):
    lane = jax.lax.broadcasted_iota(jnp.int32, (720, 128), 1)
    sub = jax.lax.broadcasted_iota(jnp.int32, (8, _IMG_NC), 0)
    for src, dst in ((a_ref, oa_ref), (b_ref, ob_ref)):
        for v in range(90):
            yo0 = 8 * v
            m = _iy_rel(yo0)
            q = 8 * (sub + yo0 + 256) - 1
            iyrel = q // 10 - _IMG_R0 - m
            fy = (q - 10 * (q // 10)).astype(jnp.float32) * 0.1
            win = src[0, m:m + 8, :]
            g0 = jnp.take_along_axis(win, iyrel, axis=0)
            g1 = jnp.take_along_axis(win, iyrel + 1, axis=0)
            v_ref[yo0:yo0 + 8, :] = g0 * (1.0 - fy) + g1 * fy
        for t in range(10):
            s = (512 * t + 2) // 5
            win = v_ref[:, s:s + 128]
            xq = 8 * (512 + 128 * t + lane) - 1
            jabs = xq // 10
            jrel = jabs - (_IMG_C0 + s)
            fx = (xq - 10 * jabs).astype(jnp.float32) * 0.1
            g0 = jnp.take_along_axis(win, jrel, axis=1)
            g1 = jnp.take_along_axis(win, jrel + 1, axis=1)
            dst[0, :, 128 * t:128 * (t + 1)] = g0 * (1.0 - fx) + g1 * fx


def _flow_body(f_ref, m_ref, of_ref, ov_ref, vx_ref, vm_ref):
    lane = jax.lax.broadcasted_iota(jnp.int32, (720, 128), 1)
    sub = jax.lax.broadcasted_iota(jnp.int32, (8, _FLW_NC), 0)
    for v in range(90):
        yo0 = 8 * v
        m = _sy_rel(yo0)
        y = sub + yo0 + 256
        r = y % 10
        sy = (8 * (y // 10) + r - (r >= 4).astype(jnp.int32)
              - (r >= 8).astype(jnp.int32) - _FLW_R0)
        idx = sy - m
        rowkeep = jnp.logical_and(r != 3, r != 7).astype(jnp.float32)
        winf = f_ref[0, m:m + 8, :]
        winm = m_ref[m:m + 8, :].astype(jnp.float32)
        vx_ref[yo0:yo0 + 8, :] = jnp.take_along_axis(winf, idx, axis=0) * (1.25 * rowkeep)
        vm_ref[yo0:yo0 + 8, :] = jnp.take_along_axis(winm, idx, axis=0) * rowkeep
    for t in range(10):
        w = _FLW_WSTART[t]
        xa = 512 + 128 * t + lane
        r = xa % 10
        sx = (8 * (xa // 10) + r - (r >= 4).astype(jnp.int32)
              - (r >= 8).astype(jnp.int32))
        jrel = sx - (_FLW_C0 + w)
        colmask = jnp.logical_and(r != 3, r != 7)
        gx = jnp.take_along_axis(vx_ref[:, w:w + 128], jrel, axis=1)
        gm = jnp.take_along_axis(vm_ref[:, w:w + 128], jrel, axis=1)
        of_ref[0, :, 128 * t:128 * (t + 1)] = gx * gm * colmask.astype(jnp.float32)
        ov_ref[:, 128 * t:128 * (t + 1)] = jnp.logical_and(gm > 0.5, colmask)


def _run_images(i1, i2):
    return pl.pallas_call(
        _img_body,
        grid=(3,),
        in_specs=[pl.BlockSpec((1, _IMG_NR, _IMG_NC), lambda c: (c, 0, 0))] * 2,
        out_specs=[pl.BlockSpec((1, 720, 1280), lambda c: (c, 0, 0))] * 2,
        out_shape=[jax.ShapeDtypeStruct((3, 720, 1280), jnp.float32)] * 2,
        scratch_shapes=[pltpu.VMEM((720, _IMG_NC), jnp.float32)],
    )(i1, i2)


def _run_flow(fl, mk):
    return pl.pallas_call(
        _flow_body,
        grid=(2,),
        in_specs=[
            pl.BlockSpec((1, _FLW_NR, _FLW_NC), lambda c: (c, 0, 0)),
            pl.BlockSpec((_FLW_NR, _FLW_NC), lambda c: (0, 0)),
        ],
        out_specs=[
            pl.BlockSpec((1, 720, 1280), lambda c: (c, 0, 0)),
            pl.BlockSpec((720, 1280), lambda c: (0, 0)),
        ],
        out_shape=[
            jax.ShapeDtypeStruct((2, 720, 1280), jnp.float32),
            jax.ShapeDtypeStruct((720, 1280), jnp.bool_),
        ],
        scratch_shapes=[
            pltpu.VMEM((720, _FLW_NC), jnp.float32),
            pltpu.VMEM((720, _FLW_NC), jnp.float32),
        ],
    )(fl, mk)


def kernel(img1, img2, flow, valid_flow_mask):
    i1 = jax.lax.slice(img1, (0, _IMG_R0, _IMG_C0),
                       (3, _IMG_R0 + _IMG_NR, _IMG_C0 + _IMG_NC))
    i2 = jax.lax.slice(img2, (0, _IMG_R0, _IMG_C0),
                       (3, _IMG_R0 + _IMG_NR, _IMG_C0 + _IMG_NC))
    fl = jax.lax.slice(flow, (0, _FLW_R0, _FLW_C0),
                       (2, _FLW_R0 + _FLW_NR, _FLW_C0 + _FLW_NC))
    mk = jax.lax.slice(valid_flow_mask, (_FLW_R0, _FLW_C0),
                       (_FLW_R0 + _FLW_NR, _FLW_C0 + _FLW_NC))
    o1, o2 = _run_images(i1, i2)
    fo, vo = _run_flow(fl, mk)
    return o1, o2, fo, vo


# speedup vs baseline: 496.1127x; 1.3444x over previous
import jax
import jax.numpy as jnp
from jax.experimental import pallas as pl
from jax.experimental.pallas import tpu as pltpu

_IMG_R0 = 200
_IMG_NR = 592
_IMG_C0 = 384
_IMG_NC = 1152
_FLW_R0 = 200
_FLW_NR = 592
_FLW_C0 = 384
_FLW_NC = 1152
_MSK_R0 = 192
_MSK_NR = 608


def _iy_rel(yo):
    return (8 * (yo + 256) - 1) // 10 - _IMG_R0


def _sy_rel(yo):
    y = yo + 256
    r = y % 10
    return 8 * (y // 10) + r - (1 if r >= 4 else 0) - (1 if r >= 8 else 0) - _FLW_R0


def _sx_rel(xo):
    xa = 512 + xo
    r = xa % 10
    return 8 * (xa // 10) + r - (1 if r >= 4 else 0) - (1 if r >= 8 else 0) - _FLW_C0


_FLW_WSTART = tuple(_sx_rel(128 * t) for t in range(10))


def _img_window_copy(hbm_ref, win_ref, sems, ch, slot, idx):
    return pltpu.make_async_copy(
        hbm_ref.at[ch, pl.ds(_IMG_R0, _IMG_NR), pl.ds(_IMG_C0, _IMG_NC)],
        win_ref.at[slot], sems.at[slot, idx])


def _img_body(a_hbm, b_hbm, oa_ref, ob_ref, wa_ref, wb_ref, v_ref, sems):
    c = pl.program_id(0)

    @pl.when(c == 0)
    def _():
        _img_window_copy(a_hbm, wa_ref, sems, 0, 0, 0).start()
        _img_window_copy(b_hbm, wb_ref, sems, 0, 0, 1).start()

    @pl.when(c + 1 < 3)
    def _():
        _img_window_copy(a_hbm, wa_ref, sems, c + 1, (c + 1) % 2, 0).start()
        _img_window_copy(b_hbm, wb_ref, sems, c + 1, (c + 1) % 2, 1).start()

    slot = c % 2
    _img_window_copy(a_hbm, wa_ref, sems, c, slot, 0).wait()
    _img_window_copy(b_hbm, wb_ref, sems, c, slot, 1).wait()

    lane = jax.lax.broadcasted_iota(jnp.int32, (720, 128), 1)
    sub = jax.lax.broadcasted_iota(jnp.int32, (8, _IMG_NC), 0)
    for src, dst in ((wa_ref, oa_ref), (wb_ref, ob_ref)):
        for v in range(90):
            yo0 = 8 * v
            m = _iy_rel(yo0)
            q = 8 * (sub + yo0 + 256) - 1
            iyrel = q // 10 - _IMG_R0 - m
            fy = (q - 10 * (q // 10)).astype(jnp.float32) * 0.1
            win = src[slot, m:m + 8, :]
            g0 = jnp.take_along_axis(win, iyrel, axis=0)
            g1 = jnp.take_along_axis(win, iyrel + 1, axis=0)
            v_ref[yo0:yo0 + 8, :] = g0 * (1.0 - fy) + g1 * fy
        for t in range(10):
            s = (8 * (512 + 128 * t) - 1) // 10 - _IMG_C0
            win = v_ref[:, s:s + 128]
            xq = 8 * (512 + 128 * t + lane) - 1
            jabs = xq // 10
            jrel = jabs - (_IMG_C0 + s)
            fx = (xq - 10 * jabs).astype(jnp.float32) * 0.1
            g0 = jnp.take_along_axis(win, jrel, axis=1)
            g1 = jnp.take_along_axis(win, jrel + 1, axis=1)
            dst[0, :, 128 * t:128 * (t + 1)] = g0 * (1.0 - fx) + g1 * fx


def _flw_window_copy(hbm_ref, win_ref, sems, ch, slot, idx):
    return pltpu.make_async_copy(
        hbm_ref.at[ch, pl.ds(_FLW_R0, _FLW_NR), pl.ds(_FLW_C0, _FLW_NC)],
        win_ref.at[slot], sems.at[slot, idx])


def _msk_window_copy(hbm_ref, win_ref, sems):
    return pltpu.make_async_copy(
        hbm_ref.at[pl.ds(_MSK_R0, _MSK_NR), pl.ds(_FLW_C0, _FLW_NC)],
        win_ref, sems.at[0, 2])


def _flow_body(f_hbm, m_hbm, of_ref, ov_ref, wf_ref, wm_ref, vx_ref, vm_ref, sems):
    c = pl.program_id(0)

    @pl.when(c == 0)
    def _():
        _flw_window_copy(f_hbm, wf_ref, sems, 0, 0, 0).start()
        _flw_window_copy(f_hbm, wf_ref, sems, 1, 1, 0).start()
        _msk_window_copy(m_hbm, wm_ref, sems).start()

    slot = c % 2
    _flw_window_copy(f_hbm, wf_ref, sems, c, slot, 0).wait()

    @pl.when(c == 0)
    def _():
        _msk_window_copy(m_hbm, wm_ref, sems).wait()

    lane = jax.lax.broadcasted_iota(jnp.int32, (720, 128), 1)
    sub = jax.lax.broadcasted_iota(jnp.int32, (8, _FLW_NC), 0)
    for v in range(90):
        yo0 = 8 * v
        m = _sy_rel(yo0)
        y = sub + yo0 + 256
        r = y % 10
        sy = (8 * (y // 10) + r - (r >= 4).astype(jnp.int32)
              - (r >= 8).astype(jnp.int32) - _FLW_R0)
        idx = sy - m
        rowkeep = jnp.logical_and(r != 3, r != 7).astype(jnp.float32)
        winf = wf_ref[slot, m:m + 8, :]
        winm = wm_ref[m + (_FLW_R0 - _MSK_R0):m + (_FLW_R0 - _MSK_R0) + 8, :].astype(jnp.float32)
        vmc = jnp.take_along_axis(winm, idx, axis=0) * rowkeep
        vm_ref[yo0:yo0 + 8, :] = vmc
        vx_ref[yo0:yo0 + 8, :] = jnp.take_along_axis(winf, idx, axis=0) * 1.25 * vmc
    for t in range(10):
        w = _FLW_WSTART[t]
        xa = 512 + 128 * t + lane
        r = xa % 10
        sx = (8 * (xa // 10) + r - (r >= 4).astype(jnp.int32)
              - (r >= 8).astype(jnp.int32))
        jrel = sx - (_FLW_C0 + w)
        colmask = jnp.logical_and(r != 3, r != 7)
        gx = jnp.take_along_axis(vx_ref[:, w:w + 128], jrel, axis=1)
        of_ref[0, :, 128 * t:128 * (t + 1)] = gx * colmask.astype(jnp.float32)

        @pl.when(c == 0)
        def _():
            gm = jnp.take_along_axis(vm_ref[:, w:w + 128], jrel, axis=1)
            ov_ref[:, 128 * t:128 * (t + 1)] = jnp.logical_and(gm > 0.5, colmask)


def _run_images(i1, i2):
    return pl.pallas_call(
        _img_body,
        grid=(3,),
        in_specs=[pl.BlockSpec(memory_space=pl.ANY)] * 2,
        out_specs=[pl.BlockSpec((1, 720, 1280), lambda c: (c, 0, 0))] * 2,
        out_shape=[jax.ShapeDtypeStruct((3, 720, 1280), jnp.float32)] * 2,
        scratch_shapes=[
            pltpu.VMEM((2, _IMG_NR, _IMG_NC), jnp.float32),
            pltpu.VMEM((2, _IMG_NR, _IMG_NC), jnp.float32),
            pltpu.VMEM((720, _IMG_NC), jnp.float32),
            pltpu.SemaphoreType.DMA((2, 2)),
        ],
    )(i1, i2)


def _run_flow(fl, mk):
    return pl.pallas_call(
        _flow_body,
        grid=(2,),
        in_specs=[pl.BlockSpec(memory_space=pl.ANY)] * 2,
        out_specs=[
            pl.BlockSpec((1, 720, 1280), lambda c: (c, 0, 0)),
            pl.BlockSpec((720, 1280), lambda c: (0, 0)),
        ],
        out_shape=[
            jax.ShapeDtypeStruct((2, 720, 1280), jnp.float32),
            jax.ShapeDtypeStruct((720, 1280), jnp.bool_),
        ],
        scratch_shapes=[
            pltpu.VMEM((2, _FLW_NR, _FLW_NC), jnp.float32),
            pltpu.VMEM((_MSK_NR, _FLW_NC), jnp.int8),
            pltpu.VMEM((720, _FLW_NC), jnp.float32),
            pltpu.VMEM((720, _FLW_NC), jnp.float32),
            pltpu.SemaphoreType.DMA((2, 3)),
        ],
    )(fl, mk)


def kernel(img1, img2, flow, valid_flow_mask):
    o1, o2 = _run_images(img1, img2)
    mk8 = valid_flow_mask.view(jnp.int8)
    fo, vo = _run_flow(flow, mk8)
    return o1, o2, fo, vo


# speedup vs baseline: 576.8788x; 1.1628x over previous
import jax
import jax.numpy as jnp
from jax.experimental import pallas as pl
from jax.experimental.pallas import tpu as pltpu

_R0 = 200
_NR = 592
_C0 = 384
_NC = 1152
_MSK_R0 = 192
_MSK_NR = 608


def _iy_rel(yo):
    return (8 * (yo + 256) - 1) // 10 - _R0


def _sy_rel(yo):
    y = yo + 256
    r = y % 10
    return 8 * (y // 10) + r - (1 if r >= 4 else 0) - (1 if r >= 8 else 0) - _R0


def _sx_rel(xo):
    xa = 512 + xo
    r = xa % 10
    return 8 * (xa // 10) + r - (1 if r >= 4 else 0) - (1 if r >= 8 else 0) - _C0


_FLW_WSTART = tuple(_sx_rel(128 * t) for t in range(10))


def _win_copy(hbm_ref, win_ref, sem, ch):
    return pltpu.make_async_copy(
        hbm_ref.at[ch, pl.ds(_R0, _NR), pl.ds(_C0, _NC)], win_ref, sem)


def _msk_copy(hbm_ref, win_ref, sem):
    return pltpu.make_async_copy(
        hbm_ref.at[pl.ds(_MSK_R0, _MSK_NR), pl.ds(_C0, _NC)], win_ref, sem)


def _body(a_hbm, b_hbm, f_hbm, m_hbm, oa_ref, ob_ref, of_ref, ov_ref,
          wa_ref, wb_ref, wf_ref, wm_ref, va_ref, vb_ref, sems):
    c = pl.program_id(0)
    lane = jax.lax.broadcasted_iota(jnp.int32, (720, 128), 1)
    sub = jax.lax.broadcasted_iota(jnp.int32, (8, _NC), 0)

    @pl.when(c == 0)
    def _():
        _win_copy(a_hbm, wa_ref.at[0], sems.at[0, 0], 0).start()
        _win_copy(b_hbm, wb_ref.at[0], sems.at[0, 1], 0).start()

    @pl.when(c < 2)
    def _():
        _win_copy(a_hbm, wa_ref.at[(c + 1) % 2], sems.at[(c + 1) % 2, 0], c + 1).start()
        _win_copy(b_hbm, wb_ref.at[(c + 1) % 2], sems.at[(c + 1) % 2, 1], c + 1).start()

    @pl.when(c == 2)
    def _():
        _win_copy(f_hbm, wf_ref.at[0], sems.at[0, 2], 0).start()
        _win_copy(f_hbm, wf_ref.at[1], sems.at[1, 2], 1).start()
        _msk_copy(m_hbm, wm_ref, sems.at[0, 3]).start()

    @pl.when(c < 3)
    def _():
        slot = c % 2
        _win_copy(a_hbm, wa_ref.at[slot], sems.at[slot, 0], c).wait()
        _win_copy(b_hbm, wb_ref.at[slot], sems.at[slot, 1], c).wait()
        for v in range(90):
            yo0 = 8 * v
            m = _iy_rel(yo0)
            q = 8 * (sub + yo0 + 256) - 1
            iyrel = q // 10 - _R0 - m
            fy = (q - 10 * (q // 10)).astype(jnp.float32) * 0.1
            wina = wa_ref[slot, m:m + 8, :]
            winb = wb_ref[slot, m:m + 8, :]
            a0 = jnp.take_along_axis(wina, iyrel, axis=0)
            b0 = jnp.take_along_axis(winb, iyrel, axis=0)
            a1 = jnp.take_along_axis(wina, iyrel + 1, axis=0)
            b1 = jnp.take_along_axis(winb, iyrel + 1, axis=0)
            va_ref[yo0:yo0 + 8, :] = a0 * (1.0 - fy) + a1 * fy
            vb_ref[yo0:yo0 + 8, :] = b0 * (1.0 - fy) + b1 * fy
        for t in range(10):
            s = (8 * (512 + 128 * t) - 1) // 10 - _C0
            xq = 8 * (512 + 128 * t + lane) - 1
            jabs = xq // 10
            jrel = jabs - (_C0 + s)
            fx = (xq - 10 * jabs).astype(jnp.float32) * 0.1
            wina = va_ref[:, s:s + 128]
            winb = vb_ref[:, s:s + 128]
            a0 = jnp.take_along_axis(wina, jrel, axis=1)
            b0 = jnp.take_along_axis(winb, jrel, axis=1)
            a1 = jnp.take_along_axis(wina, jrel + 1, axis=1)
            b1 = jnp.take_along_axis(winb, jrel + 1, axis=1)
            oa_ref[0, :, 128 * t:128 * (t + 1)] = a0 * (1.0 - fx) + a1 * fx
            ob_ref[0, :, 128 * t:128 * (t + 1)] = b0 * (1.0 - fx) + b1 * fx

    @pl.when(c >= 3)
    def _():
        ch = c - 3
        _win_copy(f_hbm, wf_ref.at[ch], sems.at[ch, 2], ch).wait()

        @pl.when(c == 3)
        def _():
            _msk_copy(m_hbm, wm_ref, sems.at[0, 3]).wait()

        mrow = _R0 - _MSK_R0
        for v in range(90):
            yo0 = 8 * v
            m = _sy_rel(yo0)
            y = sub + yo0 + 256
            r = y % 10
            sy = (8 * (y // 10) + r - (r >= 4).astype(jnp.int32)
                  - (r >= 8).astype(jnp.int32) - _R0)
            idx = sy - m
            rowkeep = jnp.logical_and(r != 3, r != 7).astype(jnp.float32)
            winf = wf_ref[ch, m:m + 8, :]
            winm = wm_ref[m + mrow:m + mrow + 8, :].astype(jnp.float32)
            vmc = jnp.take_along_axis(winm, idx, axis=0) * rowkeep
            vb_ref[yo0:yo0 + 8, :] = vmc
            va_ref[yo0:yo0 + 8, :] = jnp.take_along_axis(winf, idx, axis=0) * 1.25 * vmc
        for t in range(10):
            w = _FLW_WSTART[t]
            xa = 512 + 128 * t + lane
            r = xa % 10
            sx = (8 * (xa // 10) + r - (r >= 4).astype(jnp.int32)
                  - (r >= 8).astype(jnp.int32))
            jrel = sx - (_C0 + w)
            colmask = jnp.logical_and(r != 3, r != 7)
            gx = jnp.take_along_axis(va_ref[:, w:w + 128], jrel, axis=1)
            of_ref[0, :, 128 * t:128 * (t + 1)] = gx * colmask.astype(jnp.float32)

            @pl.when(c == 3)
            def _():
                gm = jnp.take_along_axis(vb_ref[:, w:w + 128], jrel, axis=1)
                ov_ref[:, 128 * t:128 * (t + 1)] = jnp.logical_and(gm > 0.5, colmask)


def kernel(img1, img2, flow, valid_flow_mask):
    mk8 = valid_flow_mask.view(jnp.int8)
    o1, o2, fo, vo = pl.pallas_call(
        _body,
        grid=(5,),
        in_specs=[pl.BlockSpec(memory_space=pl.ANY)] * 4,
        out_specs=[
            pl.BlockSpec((1, 720, 1280), lambda c: (jnp.minimum(c, 2), 0, 0)),
            pl.BlockSpec((1, 720, 1280), lambda c: (jnp.minimum(c, 2), 0, 0)),
            pl.BlockSpec((1, 720, 1280), lambda c: (jnp.maximum(c - 3, 0), 0, 0)),
            pl.BlockSpec((720, 1280), lambda c: (0, 0)),
        ],
        out_shape=[
            jax.ShapeDtypeStruct((3, 720, 1280), jnp.float32),
            jax.ShapeDtypeStruct((3, 720, 1280), jnp.float32),
            jax.ShapeDtypeStruct((2, 720, 1280), jnp.float32),
            jax.ShapeDtypeStruct((720, 1280), jnp.bool_),
        ],
        scratch_shapes=[
            pltpu.VMEM((2, _NR, _NC), jnp.float32),
            pltpu.VMEM((2, _NR, _NC), jnp.float32),
            pltpu.VMEM((2, _NR, _NC), jnp.float32),
            pltpu.VMEM((_MSK_NR, _NC), jnp.int8),
            pltpu.VMEM((720, _NC), jnp.float32),
            pltpu.VMEM((720, _NC), jnp.float32),
            pltpu.SemaphoreType.DMA((2, 4)),
        ],
    )(img1, img2, flow, mk8)
    return o1, o2, fo, vo
